# Initial kernel scaffold; baseline (speedup 1.0000x reference)
import jax
import jax.numpy as jnp
from jax.experimental import pallas as pl


def kernel(h, edge_index, edge_weight, W1, b1, Wmu, bmu, Wstd, bstd):
    raise NotImplementedError("write your pallas kernel here")


# speedup vs baseline: 6.8108x; 6.8108x over previous
import functools

import jax
import jax.numpy as jnp
from jax import lax
from jax.experimental import pallas as pl
from jax.experimental.pallas import tpu as pltpu
from jax.experimental.pallas import tpu_sc as plsc

N_NODES = 10000
N_EDGES = 320000
D_FEAT = 128
D_HID = 128
D_OUT = 64

NC = 2
NS = 16
L = 16
NW = NC * NS

CH = 128
CPT = 80
EPAD = NW * CPT * CH
NPAD = 10240
ROWS_PT = NPAD // NS
RB = 128

@functools.cache
def _get_mesh():
    return plsc.VectorSubcoreMesh(
        core_axis_name="c", subcore_axis_name="s",
        num_cores=NC, num_subcores=NS)


@functools.cache
def _get_deg_kernel():
    return functools.partial(
        pl.kernel,
        out_type=jax.ShapeDtypeStruct((NC, NPAD, L), jnp.float32),
        mesh=_get_mesh(),
        scratch_types=[
            pltpu.VMEM((CPT, CH), jnp.int32),
            pltpu.VMEM((CPT, CH), jnp.float32),
            pltpu.VMEM((CH, L), jnp.float32),
            pltpu.VMEM((ROWS_PT, L), jnp.float32),
            pltpu.VMEM_SHARED((NPAD, L), jnp.float32),
        ],
        compiler_params=pltpu.CompilerParams(use_tc_tiling_on_sc=False),
    )(_deg_body)


def _deg_body(row_hbm, w_hbm, out_hbm, idx_v, w_v, src_v, stage_v, acc_sh):
    c = lax.axis_index("c")
    s = lax.axis_index("s")
    wid = c * NS + s
    base = wid * CPT

    pltpu.sync_copy(row_hbm.at[pl.ds(base, CPT)], idx_v)
    pltpu.sync_copy(w_hbm.at[pl.ds(base, CPT)], w_v)

    def _z(i, _):
        stage_v[i] = jnp.zeros((L,), jnp.float32)
        return 0
    lax.fori_loop(0, ROWS_PT, _z, 0)
    pltpu.sync_copy(stage_v, acc_sh.at[pl.ds(s * ROWS_PT, ROWS_PT)])
    plsc.subcore_barrier()

    def _chunk(j, _):
        def _edges(eo, _):
            w16 = w_v[j, pl.ds(eo * L, L)]
            for u in range(L):
                src_v[eo * L + u] = jnp.full((L,), w16[u], jnp.float32)
            return 0
        lax.fori_loop(0, CH // L, _edges, 0)
        pltpu.sync_copy(src_v, acc_sh.at[idx_v.at[j]], add=True)
        return 0
    lax.fori_loop(0, CPT, _chunk, 0)
    plsc.subcore_barrier()

    for i in range(ROWS_PT // RB):
        off = s * ROWS_PT + i * RB
        pltpu.sync_copy(acc_sh.at[pl.ds(off, RB)], stage_v.at[pl.ds(0, RB)])
        pltpu.sync_copy(stage_v.at[pl.ds(0, RB)], out_hbm.at[c, pl.ds(off, RB)])


@functools.cache
def _make_prop_kernel(D):
    @functools.partial(
        pl.kernel,
        out_type=jax.ShapeDtypeStruct((NC, NPAD, D), jnp.float32),
        mesh=_get_mesh(),
        scratch_types=[
            pltpu.VMEM((CPT, CH), jnp.int32),
            pltpu.VMEM((CPT, CH), jnp.int32),
            pltpu.VMEM((CPT, CH), jnp.float32),
            pltpu.VMEM((CH, D), jnp.float32),
            pltpu.VMEM((RB, D), jnp.float32),
            pltpu.VMEM_SHARED((NPAD, D), jnp.float32),
            pltpu.SemaphoreType.DMA,
        ],
        compiler_params=pltpu.CompilerParams(use_tc_tiling_on_sc=False),
    )
    def _prop(table_hbm, row_hbm, col_hbm, w_hbm, out_hbm,
              row_v, col_v, w_v, rows_v, stage_v, acc_sh, sem):
        c = lax.axis_index("c")
        s = lax.axis_index("s")
        wid = c * NS + s
        base = wid * CPT

        pltpu.sync_copy(row_hbm.at[pl.ds(base, CPT)], row_v)
        pltpu.sync_copy(col_hbm.at[pl.ds(base, CPT)], col_v)
        pltpu.sync_copy(w_hbm.at[pl.ds(base, CPT)], w_v)

        def _z(i, _):
            for k in range(D // L):
                stage_v[i, pl.ds(k * L, L)] = jnp.zeros((L,), jnp.float32)
            return 0
        lax.fori_loop(0, RB, _z, 0)
        for i in range(ROWS_PT // RB):
            off = s * ROWS_PT + i * RB
            pltpu.sync_copy(stage_v, acc_sh.at[pl.ds(off, RB)])
        plsc.subcore_barrier()

        def _chunk(j, _):
            pltpu.async_copy(table_hbm.at[col_v.at[j]], rows_v, sem).wait()

            def _edges(eo, _):
                w16 = w_v[j, pl.ds(eo * L, L)]
                for u in range(L):
                    e = eo * L + u
                    wvec = jnp.full((L,), w16[u], jnp.float32)
                    for k in range(D // L):
                        v = rows_v[e, pl.ds(k * L, L)]
                        rows_v[e, pl.ds(k * L, L)] = v * wvec
                return 0
            lax.fori_loop(0, CH // L, _edges, 0)
            pltpu.sync_copy(rows_v, acc_sh.at[row_v.at[j]], add=True)
            return 0
        lax.fori_loop(0, CPT, _chunk, 0)
        plsc.subcore_barrier()

        for i in range(ROWS_PT // RB):
            off = s * ROWS_PT + i * RB
            pltpu.sync_copy(acc_sh.at[pl.ds(off, RB)], stage_v)
            pltpu.sync_copy(stage_v, out_hbm.at[c, pl.ds(off, RB)])

    return _prop


_BLK = 1024


def _dis_from(deg0, deg1):
    deg = deg0[:, 0:1] + deg1[:, 0:1] + 1.0
    dis = jnp.where(deg > 0, lax.rsqrt(deg), 0.0)
    return dis, dis * dis


def _mm1_body(h_ref, w_ref, d0_ref, d1_ref, hw_ref, ga_ref, gb_ref):
    hw = jnp.dot(h_ref[...], w_ref[...], preferred_element_type=jnp.float32)
    dis, _ = _dis_from(d0_ref[...], d1_ref[...])
    g = hw * dis
    hw_ref[...] = hw
    ga_ref[...] = g[:, :D_HID // 2]
    gb_ref[...] = g[:, D_HID // 2:]


def _layer1_body(p0a_ref, p1a_ref, p0b_ref, p1b_ref, hw_ref,
                 d0_ref, d1_ref, b1_ref, wmu_ref, hw2_ref, g2_ref):
    dis, dis2 = _dis_from(d0_ref[...], d1_ref[...])
    p = jnp.concatenate(
        [p0a_ref[...] + p1a_ref[...], p0b_ref[...] + p1b_ref[...]], axis=1)
    h1 = dis * p + dis2 * hw_ref[...] + b1_ref[...]
    h1 = jnp.maximum(h1, 0.0)
    hw2 = jnp.dot(h1, wmu_ref[...], preferred_element_type=jnp.float32)
    hw2_ref[...] = hw2
    g2_ref[...] = hw2 * dis


def _layer2_body(q0_ref, q1_ref, hw2_ref, d0_ref, d1_ref, bmu_ref, mu_ref):
    dis, dis2 = _dis_from(d0_ref[...], d1_ref[...])
    mu_ref[...] = (dis * (q0_ref[...] + q1_ref[...])
                   + dis2 * hw2_ref[...] + bmu_ref[...])


def _row_spec(d):
    return pl.BlockSpec((_BLK, d), lambda i: (i, 0))


def _full_spec(r, c):
    return pl.BlockSpec((r, c), lambda i: (0, 0))


def _mm1(h, W1, deg0, deg1):
    return pl.pallas_call(
        _mm1_body,
        grid=(NPAD // _BLK,),
        in_specs=[_row_spec(D_FEAT), _full_spec(D_FEAT, D_HID),
                  _row_spec(L), _row_spec(L)],
        out_specs=[_row_spec(D_HID), _row_spec(D_HID // 2),
                   _row_spec(D_HID // 2)],
        out_shape=[jax.ShapeDtypeStruct((NPAD, D_HID), jnp.float32),
                   jax.ShapeDtypeStruct((NPAD, D_HID // 2), jnp.float32),
                   jax.ShapeDtypeStruct((NPAD, D_HID // 2), jnp.float32)],
    )(h, W1, deg0, deg1)


def _layer1(ppa, ppb, hw, deg0, deg1, b1, Wmu):
    return pl.pallas_call(
        _layer1_body,
        grid=(NPAD // _BLK,),
        in_specs=[_row_spec(D_HID // 2), _row_spec(D_HID // 2),
                  _row_spec(D_HID // 2), _row_spec(D_HID // 2),
                  _row_spec(D_HID),
                  _row_spec(L), _row_spec(L),
                  _full_spec(1, D_HID), _full_spec(D_HID, D_OUT)],
        out_specs=[_row_spec(D_OUT), _row_spec(D_OUT)],
        out_shape=[jax.ShapeDtypeStruct((NPAD, D_OUT), jnp.float32),
                   jax.ShapeDtypeStruct((NPAD, D_OUT), jnp.float32)],
    )(ppa[0], ppa[1], ppb[0], ppb[1], hw, deg0, deg1, b1, Wmu)


def _layer2(q0, q1, hw2, deg0, deg1, bmu):
    return pl.pallas_call(
        _layer2_body,
        grid=(NPAD // _BLK,),
        in_specs=[_row_spec(D_OUT), _row_spec(D_OUT), _row_spec(D_OUT),
                  _row_spec(L), _row_spec(L), _full_spec(1, D_OUT)],
        out_specs=_row_spec(D_OUT),
        out_shape=jax.ShapeDtypeStruct((NPAD, D_OUT), jnp.float32),
    )(q0, q1, hw2, deg0, deg1, bmu)


@jax.jit
def kernel(h, edge_index, edge_weight, W1, b1, Wmu, bmu, Wstd, bstd):
    row = edge_index[0].astype(jnp.int32)
    col = edge_index[1].astype(jnp.int32)
    w = edge_weight.astype(jnp.float32)

    pad = EPAD - N_EDGES
    _hbm = functools.partial(
        pltpu.with_memory_space_constraint, memory_space=pltpu.HBM)
    row2d = _hbm(jnp.concatenate(
        [row, jnp.zeros((pad,), jnp.int32)]).reshape(NW * CPT, CH))
    col2d = _hbm(jnp.concatenate(
        [col, jnp.zeros((pad,), jnp.int32)]).reshape(NW * CPT, CH))
    w2d = _hbm(jnp.concatenate(
        [w, jnp.zeros((pad,), jnp.float32)]).reshape(NW * CPT, CH))

    h_pad = jnp.concatenate(
        [h, jnp.zeros((NPAD - N_NODES, D_FEAT), jnp.float32)])

    degp = _get_deg_kernel()(row2d, w2d)
    deg0, deg1 = degp[0], degp[1]

    hw1, g1a, g1b = _mm1(h_pad, W1, deg0, deg1)
    prop = _make_prop_kernel(D_HID // 2)
    ppa = prop(_hbm(g1a), row2d, col2d, w2d)
    ppb = prop(_hbm(g1b), row2d, col2d, w2d)
    hw2, g2 = _layer1(ppa, ppb, hw1, deg0, deg1,
                      b1.reshape(1, D_HID), Wmu)
    qq = _make_prop_kernel(D_OUT)(_hbm(g2), row2d, col2d, w2d)
    mu = _layer2(qq[0], qq[1], hw2, deg0, deg1, bmu.reshape(1, D_OUT))

    kl = jnp.zeros((N_NODES, D_OUT), jnp.float32)
    return (mu[:N_NODES], kl)


# speedup vs baseline: 15.3006x; 2.2465x over previous
import functools

import jax
import jax.numpy as jnp
from jax import lax
from jax.experimental import pallas as pl
from jax.experimental.pallas import tpu as pltpu
from jax.experimental.pallas import tpu_sc as plsc

N_NODES = 10000
N_EDGES = 320000
D_FEAT = 128
D_HID = 128
D_OUT = 64

NC = 2
NS = 16
L = 16
NW = NC * NS

CH = 128
CPT = 80
EPAD = NW * CPT * CH
NPAD = 10240
ROWS_PT = NPAD // NS
RB = 128

@functools.cache
def _get_mesh():
    return plsc.VectorSubcoreMesh(
        core_axis_name="c", subcore_axis_name="s",
        num_cores=NC, num_subcores=NS)


@functools.cache
def _get_deg_kernel():
    return functools.partial(
        pl.kernel,
        out_type=jax.ShapeDtypeStruct((NC, NPAD, L), jnp.float32),
        mesh=_get_mesh(),
        scratch_types=[
            pltpu.VMEM((CPT, 3, CH), jnp.int32),
            pltpu.VMEM((CH, L), jnp.float32),
            pltpu.VMEM((ROWS_PT, L), jnp.float32),
            pltpu.VMEM_SHARED((NPAD, L), jnp.float32),
        ],
        compiler_params=pltpu.CompilerParams(
            use_tc_tiling_on_sc=False, needs_layout_passes=False),
    )(_deg_body)


def _deg_body(edata_hbm, out_hbm, ebuf, src_v, stage_v, acc_sh):
    c = lax.axis_index("c")
    s = lax.axis_index("s")
    wid = c * NS + s
    base = wid * CPT

    pltpu.sync_copy(edata_hbm.at[pl.ds(base, CPT)], ebuf)

    def _z(i, _):
        stage_v[i] = jnp.zeros((L,), jnp.float32)
        return 0
    lax.fori_loop(0, ROWS_PT, _z, 0)
    pltpu.sync_copy(stage_v, acc_sh.at[pl.ds(s * ROWS_PT, ROWS_PT)])
    plsc.subcore_barrier()

    def _chunk(j, _):
        def _edges(eo, _):
            w16 = plsc.bitcast(ebuf[j, 2, pl.ds(eo * L, L)], jnp.float32)
            for u in range(L):
                src_v[eo * L + u] = jnp.full((L,), w16[u], jnp.float32)
            return 0
        lax.fori_loop(0, CH // L, _edges, 0)
        pltpu.sync_copy(src_v, acc_sh.at[ebuf.at[j, 0]], add=True)
        return 0
    lax.fori_loop(0, CPT, _chunk, 0)
    plsc.subcore_barrier()

    for i in range(ROWS_PT // RB):
        off = s * ROWS_PT + i * RB
        pltpu.sync_copy(acc_sh.at[pl.ds(off, RB)], stage_v.at[pl.ds(0, RB)])
        pltpu.sync_copy(stage_v.at[pl.ds(0, RB)], out_hbm.at[c, pl.ds(off, RB)])


NB = 4
SB = 2
CPTT = NW * CPT * CH // (NS * CH)


NG = CPTT // NB


@functools.cache
def _make_dual_prop(D):
    @functools.partial(
        pl.kernel,
        out_type=jax.ShapeDtypeStruct((NC, NPAD, D), jnp.float32),
        mesh=_get_mesh(),
        scratch_types=[
            pltpu.VMEM((2, NB, 3, CH), jnp.int32),
            pltpu.VMEM((SB, CH), jnp.int32),
            pltpu.VMEM((NB, CH, D), jnp.float32),
            pltpu.VMEM((SB, CH, D), jnp.float32),
            pltpu.VMEM((RB, D), jnp.float32),
            pltpu.VMEM_SHARED((NPAD, D), jnp.float32),
            [pltpu.SemaphoreType.DMA] * 2,
            [pltpu.SemaphoreType.DMA] * NB,
            [pltpu.SemaphoreType.DMA] * SB,
        ],
        compiler_params=pltpu.CompilerParams(
            use_tc_tiling_on_sc=False, needs_layout_passes=False),
    )
    def _prop(tab_hbm, edata_hbm, out_hbm,
              eslot, sidx, gbuf, sbuf, stage_v, acc_sh, isems, gsems, ssems):
        c = lax.axis_index("c")
        s = lax.axis_index("s")
        base = s * CPTT
        tab = tab_hbm.at[c]

        pltpu.sync_copy(edata_hbm.at[pl.ds(base, NB)], eslot.at[0])
        pltpu.sync_copy(edata_hbm.at[pl.ds(base + NB, NB)], eslot.at[1])

        def _z(i, _):
            for k in range(D // L):
                stage_v[i, pl.ds(k * L, L)] = jnp.zeros((L,), jnp.float32)
            return 0
        lax.fori_loop(0, RB, _z, 0)
        for i in range(ROWS_PT // RB):
            off = s * ROWS_PT + i * RB
            pltpu.sync_copy(stage_v, acc_sh.at[pl.ds(off, RB)])
        plsc.subcore_barrier()

        for b in range(NB):
            pltpu.async_copy(tab.at[eslot.at[0, b, 1]], gbuf.at[b], gsems[b])

        def _sgroup(sg, _):
            for ig in range(2):
                jo = sg * 2 + ig

                @pl.when(jnp.logical_and(jo >= 1, jo < NG - 1))
                def _():
                    pltpu.make_async_copy(
                        edata_hbm.at[pl.ds(base, NB)], eslot.at[1 - ig],
                        isems[1 - ig]).wait()

                for b in range(NB):
                    j = jo * NB + b
                    sb = b % SB
                    pltpu.make_async_copy(
                        tab.at[eslot.at[ig, b, 1]], gbuf.at[b],
                        gsems[b]).wait()
                    @pl.when(j >= SB)
                    def _():
                        pltpu.make_async_copy(
                            tab.at[eslot.at[ig, b, 1]], sbuf.at[sb],
                            ssems[sb]).wait()

                    def _edges(eo, _):
                        w16 = plsc.bitcast(
                            eslot[ig, b, 2, pl.ds(eo * L, L)], jnp.float32)
                        for u in range(L):
                            e = eo * L + u
                            wvec = jnp.full((L,), w16[u], jnp.float32)
                            for k in range(D // L):
                                v = gbuf[b, e, pl.ds(k * L, L)]
                                sbuf[sb, e, pl.ds(k * L, L)] = v * wvec
                        return 0
                    lax.fori_loop(0, CH // L, _edges, 0)

                    for k in range(CH // L):
                        sidx[sb, pl.ds(k * L, L)] = (
                            eslot[ig, b, 0, pl.ds(k * L, L)])
                    pltpu.async_copy(
                        sbuf.at[sb], acc_sh.at[sidx.at[sb]],
                        ssems[sb], add=True)

                    @pl.when(jo < NG - 1)
                    def _():
                        pltpu.async_copy(
                            tab.at[eslot.at[1 - ig, b, 1]], gbuf.at[b],
                            gsems[b])

                @pl.when(jo < NG - 2)
                def _():
                    pltpu.async_copy(
                        edata_hbm.at[pl.ds(base + (jo + 2) * NB, NB)],
                        eslot.at[ig], isems[ig])
            return 0
        lax.fori_loop(0, NG // 2, _sgroup, 0)

        for sb in range(SB):
            pltpu.make_async_copy(
                tab.at[eslot.at[0, 0, 1]], sbuf.at[sb], ssems[sb]).wait()
        plsc.subcore_barrier()

        for i in range(ROWS_PT // RB):
            off = s * ROWS_PT + i * RB
            pltpu.sync_copy(acc_sh.at[pl.ds(off, RB)], stage_v)
            pltpu.sync_copy(stage_v, out_hbm.at[c, pl.ds(off, RB)])

    return _prop


_BLK = 1024


def _dis_from(deg0, deg1):
    deg = deg0[:, 0:1] + deg1[:, 0:1] + 1.0
    dis = jnp.where(deg > 0, lax.rsqrt(deg), 0.0)
    return dis, dis * dis


def _mm1_body(h_ref, w_ref, d0_ref, d1_ref, hw_ref, g_ref):
    hw = jnp.dot(h_ref[...], w_ref[...], preferred_element_type=jnp.float32)
    dis, _ = _dis_from(d0_ref[...], d1_ref[...])
    g = hw * dis
    hw_ref[...] = hw
    g_ref[0] = g[:, :D_HID // 2]
    g_ref[1] = g[:, D_HID // 2:]


def _layer1_body(pp_ref, hw_ref, d0_ref, d1_ref, b1_ref, wmu_ref,
                 hw2_ref, g2_ref):
    dis, dis2 = _dis_from(d0_ref[...], d1_ref[...])
    p = jnp.concatenate([pp_ref[0], pp_ref[1]], axis=1)
    h1 = dis * p + dis2 * hw_ref[...] + b1_ref[...]
    h1 = jnp.maximum(h1, 0.0)
    hw2 = jnp.dot(h1, wmu_ref[...], preferred_element_type=jnp.float32)
    g2 = hw2 * dis
    hw2_ref[...] = hw2
    g2_ref[0] = g2[:, :D_OUT // 2]
    g2_ref[1] = g2[:, D_OUT // 2:]


def _layer2_body(qq_ref, hw2_ref, d0_ref, d1_ref, bmu_ref, mu_ref):
    dis, dis2 = _dis_from(d0_ref[...], d1_ref[...])
    q = jnp.concatenate([qq_ref[0], qq_ref[1]], axis=1)
    mu_ref[...] = dis * q + dis2 * hw2_ref[...] + bmu_ref[...]


def _row_spec(d):
    return pl.BlockSpec((_BLK, d), lambda i: (i, 0))


def _stack_spec(d):
    return pl.BlockSpec((NC, _BLK, d), lambda i: (0, i, 0))


def _full_spec(r, c):
    return pl.BlockSpec((r, c), lambda i: (0, 0))


def _mm1(h, W1, deg0, deg1):
    return pl.pallas_call(
        _mm1_body,
        grid=(NPAD // _BLK,),
        in_specs=[_row_spec(D_FEAT), _full_spec(D_FEAT, D_HID),
                  _row_spec(L), _row_spec(L)],
        out_specs=[_row_spec(D_HID), _stack_spec(D_HID // 2)],
        out_shape=[jax.ShapeDtypeStruct((NPAD, D_HID), jnp.float32),
                   jax.ShapeDtypeStruct((NC, NPAD, D_HID // 2), jnp.float32)],
    )(h, W1, deg0, deg1)


def _layer1(pp, hw, deg0, deg1, b1, Wmu):
    return pl.pallas_call(
        _layer1_body,
        grid=(NPAD // _BLK,),
        in_specs=[_stack_spec(D_HID // 2), _row_spec(D_HID),
                  _row_spec(L), _row_spec(L),
                  _full_spec(1, D_HID), _full_spec(D_HID, D_OUT)],
        out_specs=[_row_spec(D_OUT), _stack_spec(D_OUT // 2)],
        out_shape=[jax.ShapeDtypeStruct((NPAD, D_OUT), jnp.float32),
                   jax.ShapeDtypeStruct((NC, NPAD, D_OUT // 2), jnp.float32)],
    )(pp, hw, deg0, deg1, b1, Wmu)


def _layer2(qq, hw2, deg0, deg1, bmu):
    return pl.pallas_call(
        _layer2_body,
        grid=(NPAD // _BLK,),
        in_specs=[_stack_spec(D_OUT // 2), _row_spec(D_OUT),
                  _row_spec(L), _row_spec(L), _full_spec(1, D_OUT)],
        out_specs=_row_spec(D_OUT),
        out_shape=jax.ShapeDtypeStruct((NPAD, D_OUT), jnp.float32),
    )(qq, hw2, deg0, deg1, bmu)


@jax.jit
def kernel(h, edge_index, edge_weight, W1, b1, Wmu, bmu, Wstd, bstd):
    row = edge_index[0].astype(jnp.int32)
    col = edge_index[1].astype(jnp.int32)
    w = edge_weight.astype(jnp.float32)

    pad = EPAD - N_EDGES
    _hbm = functools.partial(
        pltpu.with_memory_space_constraint, memory_space=pltpu.HBM)
    row_p = jnp.concatenate(
        [row, jnp.zeros((pad,), jnp.int32)]).reshape(NW * CPT, CH)
    col_p = jnp.concatenate(
        [col, jnp.zeros((pad,), jnp.int32)]).reshape(NW * CPT, CH)
    wbits = lax.bitcast_convert_type(jnp.concatenate(
        [w, jnp.zeros((pad,), jnp.float32)]), jnp.int32).reshape(NW * CPT, CH)
    edata = _hbm(jnp.stack([row_p, col_p, wbits], axis=1))

    h_pad = jnp.concatenate(
        [h, jnp.zeros((NPAD - N_NODES, D_FEAT), jnp.float32)])

    degp = _get_deg_kernel()(edata)
    deg0, deg1 = degp[0], degp[1]

    hw1, g1s = _mm1(h_pad, W1, deg0, deg1)
    pp = _make_dual_prop(D_HID // 2)(_hbm(g1s), edata)
    hw2, g2s = _layer1(pp, hw1, deg0, deg1, b1.reshape(1, D_HID), Wmu)
    qq = _make_dual_prop(D_OUT // 2)(_hbm(g2s), edata)
    mu = _layer2(qq, hw2, deg0, deg1, bmu.reshape(1, D_OUT))

    kl = jnp.zeros((N_NODES, D_OUT), jnp.float32)
    return (mu[:N_NODES], kl)
